# Initial kernel scaffold; baseline (speedup 1.0000x reference)
import jax
import jax.numpy as jnp
from jax.experimental import pallas as pl


def kernel(cell, x, z, num_atoms, W_in, W_edge, W_msg, b_msg, W_upd, W_force, W_stress):
    raise NotImplementedError("write your pallas kernel here")


# speedup vs baseline: 3.1268x; 3.1268x over previous
import jax
import jax.numpy as jnp
from jax import lax
from jax.experimental import pallas as pl

_K = 32
_CUT = 5.0


def _silu(v):
    return v * (1.0 / (1.0 + jnp.exp(-v)))


def _make_body(A, F, RBF, NB):
    AA = A * A

    def body(x_ref, z_ref, cell_ref, cinv_ref, win_ref, wedge_ref,
             wmsg_ref, bmsg_ref, wupd_ref, wfs_ref,
             xout_ref, traj_ref, s_ref):
        xb = x_ref[0]
        cellm = cell_ref[0]
        cinv = cinv_ref[0]

        df = xb[:, None, :] - xb[None, :, :]
        df = df - jnp.round(df)
        df2 = df.reshape(AA, 3)
        dc = jnp.dot(df2, cellm, preferred_element_type=jnp.float32)
        dist = jnp.sqrt(jnp.sum(dc * dc, axis=1, keepdims=True) + 1e-12)
        dsq = dist.reshape(A, A)
        ii = lax.broadcasted_iota(jnp.int32, (A, A), 0)
        jj = lax.broadcasted_iota(jnp.int32, (A, A), 1)
        disteye = jnp.where(ii == jj, dsq + 1e6, dsq)

        d_ij = disteye[:, :, None]
        d_ik = disteye[:, None, :]
        jx = lax.broadcasted_iota(jnp.int32, (A, A, A), 1)
        kx = lax.broadcasted_iota(jnp.int32, (A, A, A), 2)
        sel = (d_ik < d_ij) | ((d_ik == d_ij) & (kx < jx))
        rank = jnp.sum(sel.astype(jnp.float32), axis=2)
        maskf = jnp.where(rank < float(_K), 1.0, 0.0).astype(jnp.float32)
        maskc = maskf.reshape(AA, 1)

        centers = (lax.broadcasted_iota(jnp.int32, (1, RBF), 1)
                   .astype(jnp.float32) * (_CUT / (RBF - 1)))
        delta = dist - centers
        rbf = jnp.exp(-10.0 * delta * delta)
        e = _silu(jnp.dot(rbf, wedge_ref[...], preferred_element_type=jnp.float32))

        w2cat = jnp.concatenate([wmsg_ref[t, F:, :] for t in range(NB)], axis=1)
        ew2 = jnp.dot(e, w2cat, preferred_element_type=jnp.float32)

        h = jnp.dot(z_ref[0], win_ref[...], preferred_element_type=jnp.float32)
        inv_d = maskc / (dist + 1e-9)
        xcur = xb
        stress = jnp.zeros((3, 3), dtype=jnp.float32)
        for t in range(NB):
            hw1 = jnp.dot(h, wmsg_ref[t, :F, :], preferred_element_type=jnp.float32)
            pre = (ew2[:, t * F:(t + 1) * F].reshape(A, A, F)
                   + hw1[None, :, :] + bmsg_ref[t].reshape(1, 1, F))
            m = _silu(pre)
            agg = jnp.sum(m * maskf[:, :, None], axis=1)
            h = h + jnp.tanh(jnp.dot(agg, wupd_ref[t], preferred_element_type=jnp.float32))
            fsss = jnp.dot(m.reshape(AA, F), wfs_ref[...],
                           preferred_element_type=jnp.float32)
            g = fsss * inv_d
            disp = jnp.sum((g[:, 0:1] * dc).reshape(A, A, 3), axis=1)
            xcur = xcur + jnp.dot(disp, cinv, preferred_element_type=jnp.float32)
            traj_ref[t, 0] = xcur
            gsdc = g[:, 1:2] * dc
            stress = stress + jnp.sum(dc[:, :, None] * gsdc[:, None, :], axis=0)
        xout_ref[0] = xcur
        s_ref[0] = stress

    return body


def kernel(cell, x, z, num_atoms, W_in, W_edge, W_msg, b_msg, W_upd,
           W_force, W_stress):
    B = cell.shape[0]
    N = x.shape[0]
    A = N // B
    F = z.shape[1]
    RBF = W_edge.shape[0]
    NB = W_msg.shape[0]

    xb = x.reshape(B, A, 3)
    zb = z.reshape(B, A, F)
    cinv = jnp.linalg.inv(cell)
    wfs = jnp.concatenate([W_force, W_stress], axis=1)

    xout, traj, stress = pl.pallas_call(
        _make_body(A, F, RBF, NB),
        grid=(B,),
        in_specs=[
            pl.BlockSpec((1, A, 3), lambda b: (b, 0, 0)),
            pl.BlockSpec((1, A, F), lambda b: (b, 0, 0)),
            pl.BlockSpec((1, 3, 3), lambda b: (b, 0, 0)),
            pl.BlockSpec((1, 3, 3), lambda b: (b, 0, 0)),
            pl.BlockSpec((F, F), lambda b: (0, 0)),
            pl.BlockSpec((RBF, F), lambda b: (0, 0)),
            pl.BlockSpec((NB, 2 * F, F), lambda b: (0, 0, 0)),
            pl.BlockSpec((NB, F), lambda b: (0, 0)),
            pl.BlockSpec((NB, F, F), lambda b: (0, 0, 0)),
            pl.BlockSpec((F, 2), lambda b: (0, 0)),
        ],
        out_specs=[
            pl.BlockSpec((1, A, 3), lambda b: (b, 0, 0)),
            pl.BlockSpec((NB, 1, A, 3), lambda b: (0, b, 0, 0)),
            pl.BlockSpec((1, 3, 3), lambda b: (b, 0, 0)),
        ],
        out_shape=[
            jax.ShapeDtypeStruct((B, A, 3), jnp.float32),
            jax.ShapeDtypeStruct((NB, B, A, 3), jnp.float32),
            jax.ShapeDtypeStruct((B, 3, 3), jnp.float32),
        ],
    )(xb, zb, cell, cinv, W_in, W_edge, W_msg, b_msg, W_upd, wfs)

    x_prime = xout.reshape(N, 3)
    x_traj = traj.reshape(NB, N, 3)
    rho_prime = 0.5 * (stress + stress.transpose(0, 2, 1))
    return (x_prime, x_traj, rho_prime)
